# baseline (speedup 1.0000x reference)
import functools

import jax
import jax.numpy as jnp
from jax import lax
from jax.experimental import pallas as pl
from jax.experimental.pallas import tpu as pltpu
from jax.experimental.pallas import tpu_sc as plsc

D = 64
DP = 128
NW = 32
HALF_A = 104
HALF_B = 96
NBUF = 2
TBLK = 2048


def _transpose_pad(tokT):
    V = tokT.shape[1]
    grid = (V + TBLK - 1) // TBLK

    def body(in_ref, out_ref):
        ident = jnp.where(
            jax.lax.broadcasted_iota(jnp.int32, (D, D), 0)
            == jax.lax.broadcasted_iota(jnp.int32, (D, D), 1),
            1.0, 0.0)
        t = jax.lax.dot_general(
            in_ref[...], ident,
            dimension_numbers=(((0,), (0,)), ((), ())),
            preferred_element_type=jnp.float32,
        )
        out_ref[...] = jnp.concatenate([t, t], axis=1)

    return pl.pallas_call(
        body,
        grid=grid,
        in_specs=[pl.BlockSpec((D, TBLK), lambda g: (0, g))],
        out_specs=pl.BlockSpec((TBLK, DP), lambda g: (g, 0)),
        out_shape=jax.ShapeDtypeStruct((V, DP), jnp.float32),
    )(tokT)


@jax.jit
def kernel(x, token_table, pos_table):
    B, S = x.shape
    assert S == HALF_A + HALF_B and D == token_table.shape[1]
    spw = B // NW
    assert spw * NW == B and spw % NBUF == 0

    tok128 = _transpose_pad(token_table.T)

    mesh = plsc.VectorSubcoreMesh(core_axis_name="c", subcore_axis_name="s")

    @functools.partial(
        pl.kernel,
        out_type=jax.ShapeDtypeStruct((B, S, DP), jnp.float32),
        mesh=mesh,
        compiler_params=pltpu.CompilerParams(use_tc_tiling_on_sc=False),
        scratch_types=[
            pltpu.VMEM((spw, S), jnp.int32),
            pltpu.VMEM_SHARED((S, DP), jnp.float32),
            pltpu.VMEM((NBUF, S, DP), jnp.float32),
        ] + [pltpu.SemaphoreType.DMA] * (2 * NBUF + 1),
    )
    def embed(x_hbm, tok_hbm, pos_hbm, out_hbm,
              idx_v, pos_sh, rows_v, *sems):
        wid = lax.axis_index("s") * 2 + lax.axis_index("c")
        seq0 = wid * spw
        isems = sems[:NBUF]
        osems = sems[NBUF:2 * NBUF]
        gsem = sems[2 * NBUF]

        @pl.when(lax.axis_index("s") == 0)
        def _():
            pltpu.sync_copy(pos_hbm, pos_sh)

        pltpu.sync_copy(x_hbm.at[pl.ds(seq0, spw)], idx_v)
        plsc.subcore_barrier()

        pltpu.async_copy(pos_sh, rows_v.at[0], isems[0])

        def step(j, slot):
            buf = rows_v.at[slot]
            nslot = (slot + 1) % NBUF
            nbuf = rows_v.at[nslot]
            pltpu.make_async_copy(pos_sh, buf, isems[slot]).wait()
            c0 = pltpu.async_copy(
                tok_hbm.at[idx_v.at[j, pl.ds(0, HALF_A)]],
                buf.at[pl.ds(0, HALF_A)], gsem, add=True)
            c1 = pltpu.async_copy(
                tok_hbm.at[idx_v.at[j, pl.ds(HALF_A, HALF_B)]],
                buf.at[pl.ds(HALF_A, HALF_B)], gsem, add=True)

            @pl.when(j + 1 < spw)
            def _():
                @pl.when(j + 1 >= NBUF)
                def _():
                    pltpu.make_async_copy(nbuf, out_hbm.at[seq0 + j],
                                          osems[nslot]).wait()
                pltpu.async_copy(pos_sh, nbuf, isems[nslot])

            c0.wait()
            c1.wait()
            pltpu.async_copy(buf, out_hbm.at[seq0 + j], osems[slot])

        def outer(g, carry):
            for b in range(NBUF):
                step(g * NBUF + b, b)
            return carry

        lax.fori_loop(0, spw // NBUF, outer, 0)
        for j in range(spw - NBUF, spw):
            slot = j % NBUF
            pltpu.make_async_copy(
                rows_v.at[slot], out_hbm.at[seq0 + j], osems[slot]
            ).wait()

    pos128 = jnp.pad(pos_table, ((0, 0), (0, DP - D)))
    out128 = embed(x, tok128, pos128)
    return out128[:, :, :D]


# speedup vs baseline: 1.0152x; 1.0152x over previous
import functools

import jax
import jax.numpy as jnp
from jax import lax
from jax.experimental import pallas as pl
from jax.experimental.pallas import tpu as pltpu
from jax.experimental.pallas import tpu_sc as plsc

D = 64
DP = 128
NW = 32
HALF_A = 104
HALF_B = 96
NBUF = 2
TBLK = 8192


def _transpose_pad(tokT):
    V = tokT.shape[1]
    grid = (V + TBLK - 1) // TBLK

    def body(in_ref, out_ref):
        ident = jnp.where(
            jax.lax.broadcasted_iota(jnp.int32, (D, D), 0)
            == jax.lax.broadcasted_iota(jnp.int32, (D, D), 1),
            1.0, 0.0)
        t = jax.lax.dot_general(
            in_ref[...], ident,
            dimension_numbers=(((0,), (0,)), ((), ())),
            preferred_element_type=jnp.float32,
            precision=jax.lax.Precision.HIGHEST,
        )
        out_ref[...] = jnp.concatenate([t, t], axis=1)

    return pl.pallas_call(
        body,
        grid=grid,
        in_specs=[pl.BlockSpec((D, TBLK), lambda g: (0, g))],
        out_specs=pl.BlockSpec((TBLK, DP), lambda g: (g, 0)),
        out_shape=jax.ShapeDtypeStruct((V, DP), jnp.float32),
    )(tokT)


@jax.jit
def kernel(x, token_table, pos_table):
    B, S = x.shape
    assert S == HALF_A + HALF_B and D == token_table.shape[1]
    spw = B // NW
    assert spw * NW == B and spw % NBUF == 0

    tok128 = _transpose_pad(token_table.T)

    mesh = plsc.VectorSubcoreMesh(core_axis_name="c", subcore_axis_name="s")

    @functools.partial(
        pl.kernel,
        out_type=jax.ShapeDtypeStruct((B, S, DP), jnp.float32),
        mesh=mesh,
        compiler_params=pltpu.CompilerParams(use_tc_tiling_on_sc=False),
        scratch_types=[
            pltpu.VMEM((spw, S), jnp.int32),
            pltpu.VMEM_SHARED((S, D), jnp.float32),
            pltpu.VMEM((NBUF, S, DP), jnp.float32),
        ] + [pltpu.SemaphoreType.DMA] * (2 * NBUF + 1),
    )
    def embed(x_hbm, tok_hbm, pos_hbm, out_hbm,
              idx_v, pos_sh, rows_v, *sems):
        wid = lax.axis_index("s") * 2 + lax.axis_index("c")
        seq0 = wid * spw
        isems = sems[:NBUF]
        osems = sems[NBUF:2 * NBUF]
        gsem = sems[2 * NBUF]

        @pl.when(lax.axis_index("s") == 0)
        def _():
            pltpu.sync_copy(pos_hbm, pos_sh)

        pltpu.sync_copy(x_hbm.at[pl.ds(seq0, spw)], idx_v)
        plsc.subcore_barrier()

        pltpu.async_copy(pos_sh, rows_v.at[0, :, pl.ds(0, D)], isems[0])

        def step(j, slot):
            buf = rows_v.at[slot]
            buf_thin = rows_v.at[slot, :, pl.ds(0, D)]
            nslot = (slot + 1) % NBUF
            nbuf_thin = rows_v.at[nslot, :, pl.ds(0, D)]
            pltpu.make_async_copy(pos_sh, buf_thin, isems[slot]).wait()
            c0 = pltpu.async_copy(
                tok_hbm.at[idx_v.at[j, pl.ds(0, HALF_A)]],
                buf.at[pl.ds(0, HALF_A)], gsem, add=True)
            c1 = pltpu.async_copy(
                tok_hbm.at[idx_v.at[j, pl.ds(HALF_A, HALF_B)]],
                buf.at[pl.ds(HALF_A, HALF_B)], gsem, add=True)

            @pl.when(j + 1 < spw)
            def _():
                @pl.when(j + 1 >= NBUF)
                def _():
                    pltpu.make_async_copy(nbuf_thin,
                                          out_hbm.at[seq0 + j, :, pl.ds(0, D)],
                                          osems[nslot]).wait()
                pltpu.async_copy(pos_sh, nbuf_thin, isems[nslot])

            c0.wait()
            c1.wait()
            pltpu.async_copy(buf_thin, out_hbm.at[seq0 + j, :, pl.ds(0, D)],
                             osems[slot])

        def outer(g, carry):
            for b in range(NBUF):
                step(g * NBUF + b, b)
            return carry

        lax.fori_loop(0, spw // NBUF, outer, 0)
        for j in range(spw - NBUF, spw):
            slot = j % NBUF
            pltpu.make_async_copy(
                rows_v.at[slot, :, pl.ds(0, D)],
                out_hbm.at[seq0 + j, :, pl.ds(0, D)], osems[slot]
            ).wait()

    out128 = embed(x, tok128, pos_table)
    return out128[:, :, :D]


# speedup vs baseline: 1.3350x; 1.3151x over previous
import functools

import jax
import jax.numpy as jnp
from jax import lax
from jax.experimental import pallas as pl
from jax.experimental.pallas import tpu as pltpu
from jax.experimental.pallas import tpu_sc as plsc

D = 64
DP = 128
NW = 32
HALF_A = 104
HALF_B = 96
NBUF = 4
TBLK = 8192


def _transpose_pad(tokT):
    V = tokT.shape[1]
    grid = (V + TBLK - 1) // TBLK

    def body(in_ref, out_ref):
        t = jnp.transpose(in_ref[...], (1, 0))
        out_ref[...] = jnp.concatenate([t, t], axis=1)

    return pl.pallas_call(
        body,
        grid=grid,
        in_specs=[pl.BlockSpec((D, TBLK), lambda g: (0, g))],
        out_specs=pl.BlockSpec((TBLK, DP), lambda g: (g, 0)),
        out_shape=jax.ShapeDtypeStruct((V, DP), jnp.float32),
    )(tokT)


@jax.jit
def kernel(x, token_table, pos_table):
    B, S = x.shape
    assert S == HALF_A + HALF_B and D == token_table.shape[1]
    spw = B // NW
    assert spw * NW == B and spw % NBUF == 0

    tok128 = _transpose_pad(token_table.T)

    mesh = plsc.VectorSubcoreMesh(core_axis_name="c", subcore_axis_name="s")

    @functools.partial(
        pl.kernel,
        out_type=jax.ShapeDtypeStruct((B, S, DP), jnp.float32),
        mesh=mesh,
        compiler_params=pltpu.CompilerParams(use_tc_tiling_on_sc=False),
        scratch_types=[
            pltpu.VMEM((spw, S), jnp.int32),
            pltpu.VMEM_SHARED((S, D), jnp.float32),
            pltpu.VMEM((NBUF, S, DP), jnp.float32),
        ] + [pltpu.SemaphoreType.DMA] * (3 * NBUF),
    )
    def embed(x_hbm, tok_hbm, pos_hbm, out_hbm,
              idx_v, pos_sh, rows_v, *sems):
        wid = lax.axis_index("s") * 2 + lax.axis_index("c")
        seq0 = wid * spw
        isems = sems[:NBUF]
        osems = sems[NBUF:2 * NBUF]
        gsems = sems[2 * NBUF:3 * NBUF]

        @pl.when(lax.axis_index("s") == 0)
        def _():
            pltpu.sync_copy(pos_hbm, pos_sh)

        pltpu.sync_copy(x_hbm.at[pl.ds(seq0, spw)], idx_v)
        plsc.subcore_barrier()

        pltpu.async_copy(pos_sh, rows_v.at[0, :, pl.ds(0, D)], isems[0])

        def thin(slot):
            return rows_v.at[slot, :, pl.ds(0, D)]

        def out_thin(j):
            return out_hbm.at[seq0 + j, :, pl.ds(0, D)]

        def gwait(j, slot):
            buf = rows_v.at[slot]
            pltpu.make_async_copy(
                tok_hbm.at[idx_v.at[j, pl.ds(0, HALF_A)]],
                buf.at[pl.ds(0, HALF_A)], gsems[slot]).wait()
            pltpu.make_async_copy(
                tok_hbm.at[idx_v.at[j, pl.ds(HALF_A, HALF_B)]],
                buf.at[pl.ds(HALF_A, HALF_B)], gsems[slot]).wait()
            pltpu.async_copy(thin(slot), out_thin(j), osems[slot])

        def step(j, slot):
            buf = rows_v.at[slot]
            nslot = (slot + 1) % NBUF
            pltpu.make_async_copy(pos_sh, thin(slot), isems[slot]).wait()
            pltpu.async_copy(
                tok_hbm.at[idx_v.at[j, pl.ds(0, HALF_A)]],
                buf.at[pl.ds(0, HALF_A)], gsems[slot], add=True)
            pltpu.async_copy(
                tok_hbm.at[idx_v.at[j, pl.ds(HALF_A, HALF_B)]],
                buf.at[pl.ds(HALF_A, HALF_B)], gsems[slot], add=True)

            @pl.when(j >= 1)
            def _():
                gwait(j - 1, (slot - 1) % NBUF)

            @pl.when(j + 1 < spw)
            def _():
                @pl.when(j + 1 >= NBUF)
                def _():
                    pltpu.make_async_copy(thin(nslot), out_thin(j),
                                          osems[nslot]).wait()
                pltpu.async_copy(pos_sh, thin(nslot), isems[nslot])

        def outer(g, carry):
            for b in range(NBUF):
                step(g * NBUF + b, b)
            return carry

        lax.fori_loop(0, spw // NBUF, outer, 0)
        gwait(spw - 1, (spw - 1) % NBUF)
        for j in range(spw - NBUF, spw):
            pltpu.make_async_copy(
                thin(j % NBUF), out_thin(j), osems[j % NBUF]
            ).wait()

    out128 = embed(x, tok128, pos_table)
    return out128[:, :, :D]


# speedup vs baseline: 1.4391x; 1.0779x over previous
import functools

import jax
import jax.numpy as jnp
from jax import lax
from jax.experimental import pallas as pl
from jax.experimental.pallas import tpu as pltpu
from jax.experimental.pallas import tpu_sc as plsc

D = 64
DP = 128
NW = 32
HALF_A = 104
HALF_B = 96
NBUF = 4
TBLK = 8192


def _transpose_pad(tokT):
    V = tokT.shape[1]
    grid = (V + TBLK - 1) // TBLK

    def body(in_ref, out_ref):
        t = jnp.transpose(in_ref[...], (1, 0))
        out_ref[:, 0:D] = t

    return pl.pallas_call(
        body,
        grid=grid,
        in_specs=[pl.BlockSpec((D, TBLK), lambda g: (0, g))],
        out_specs=pl.BlockSpec((TBLK, DP), lambda g: (g, 0)),
        out_shape=jax.ShapeDtypeStruct((V, DP), jnp.float32),
    )(tokT)


@jax.jit
def kernel(x, token_table, pos_table):
    B, S = x.shape
    assert S == HALF_A + HALF_B and D == token_table.shape[1]
    spw = B // NW
    assert spw * NW == B and spw % NBUF == 0

    tok128 = _transpose_pad(token_table.T)

    mesh = plsc.VectorSubcoreMesh(core_axis_name="c", subcore_axis_name="s")

    @functools.partial(
        pl.kernel,
        out_type=jax.ShapeDtypeStruct((B, S, DP), jnp.float32),
        mesh=mesh,
        compiler_params=pltpu.CompilerParams(use_tc_tiling_on_sc=False),
        scratch_types=[
            pltpu.VMEM((spw, S), jnp.int32),
            pltpu.VMEM_SHARED((S, D), jnp.float32),
            pltpu.VMEM((NBUF, S, DP), jnp.float32),
        ] + [pltpu.SemaphoreType.DMA] * (3 * NBUF),
    )
    def embed(x_hbm, tok_hbm, pos_hbm, out_hbm,
              idx_v, pos_sh, rows_v, *sems):
        wid = lax.axis_index("s") * 2 + lax.axis_index("c")
        seq0 = wid * spw
        isems = sems[:NBUF]
        osems = sems[NBUF:2 * NBUF]
        gsems = sems[2 * NBUF:3 * NBUF]

        @pl.when(lax.axis_index("s") == 0)
        def _():
            pltpu.sync_copy(pos_hbm, pos_sh)

        pltpu.sync_copy(x_hbm.at[pl.ds(seq0, spw)], idx_v)
        plsc.subcore_barrier()

        pltpu.async_copy(pos_sh, rows_v.at[0, :, pl.ds(0, D)], isems[0])

        def thin(slot):
            return rows_v.at[slot, :, pl.ds(0, D)]

        def out_thin(j):
            return out_hbm.at[seq0 + j, :, pl.ds(0, D)]

        def gwait(j, slot):
            buf = rows_v.at[slot]
            pltpu.make_async_copy(
                tok_hbm.at[idx_v.at[j, pl.ds(0, HALF_A)]],
                buf.at[pl.ds(0, HALF_A)], gsems[slot]).wait()
            pltpu.make_async_copy(
                tok_hbm.at[idx_v.at[j, pl.ds(HALF_A, HALF_B)]],
                buf.at[pl.ds(HALF_A, HALF_B)], gsems[slot]).wait()
            pltpu.async_copy(thin(slot), out_thin(j), osems[slot])

        def step(j, slot):
            buf = rows_v.at[slot]
            nslot = (slot + 1) % NBUF
            pltpu.make_async_copy(pos_sh, thin(slot), isems[slot]).wait()
            pltpu.async_copy(
                tok_hbm.at[idx_v.at[j, pl.ds(0, HALF_A)]],
                buf.at[pl.ds(0, HALF_A)], gsems[slot], add=True)
            pltpu.async_copy(
                tok_hbm.at[idx_v.at[j, pl.ds(HALF_A, HALF_B)]],
                buf.at[pl.ds(HALF_A, HALF_B)], gsems[slot], add=True)

            @pl.when(j >= 1)
            def _():
                gwait(j - 1, (slot - 1) % NBUF)

            @pl.when(j + 1 < spw)
            def _():
                @pl.when(j + 1 >= NBUF)
                def _():
                    pltpu.make_async_copy(thin(nslot), out_thin(j),
                                          osems[nslot]).wait()
                pltpu.async_copy(pos_sh, thin(nslot), isems[nslot])

        def outer(g, carry):
            for b in range(NBUF):
                step(g * NBUF + b, b)
            return carry

        lax.fori_loop(0, spw // NBUF, outer, 0)
        gwait(spw - 1, (spw - 1) % NBUF)
        for j in range(spw - NBUF, spw):
            pltpu.make_async_copy(
                thin(j % NBUF), out_thin(j), osems[j % NBUF]
            ).wait()

    out128 = embed(x, tok128, pos_table)
    return out128[:, :, :D]


# speedup vs baseline: 1.4785x; 1.0274x over previous
import functools

import jax
import jax.numpy as jnp
from jax import lax
from jax.experimental import pallas as pl
from jax.experimental.pallas import tpu as pltpu
from jax.experimental.pallas import tpu_sc as plsc

D = 64
DP = 128
NW = 32
HALF_A = 104
HALF_B = 96
NBUF = 4
TBLK = 16384


def _transpose_pad(tokT):
    V = tokT.shape[1]
    grid = (V + TBLK - 1) // TBLK

    def body(in_ref, out_ref):
        t = jnp.transpose(in_ref[...], (1, 0))
        out_ref[:, 0:D] = t

    return pl.pallas_call(
        body,
        grid=grid,
        in_specs=[pl.BlockSpec((D, TBLK), lambda g: (0, g))],
        out_specs=pl.BlockSpec((TBLK, DP), lambda g: (g, 0)),
        out_shape=jax.ShapeDtypeStruct((V, DP), jnp.float32),
    )(tokT)


@jax.jit
def kernel(x, token_table, pos_table):
    B, S = x.shape
    assert S == HALF_A + HALF_B and D == token_table.shape[1]
    spw = B // NW
    assert spw * NW == B and spw % NBUF == 0

    tok128 = _transpose_pad(token_table.T)

    mesh = plsc.VectorSubcoreMesh(core_axis_name="c", subcore_axis_name="s")

    @functools.partial(
        pl.kernel,
        out_type=jax.ShapeDtypeStruct((B, S, DP), jnp.float32),
        mesh=mesh,
        compiler_params=pltpu.CompilerParams(use_tc_tiling_on_sc=False),
        scratch_types=[
            pltpu.VMEM((spw, S), jnp.int32),
            pltpu.VMEM_SHARED((S, D), jnp.float32),
            pltpu.VMEM((NBUF, S, DP), jnp.float32),
        ] + [pltpu.SemaphoreType.DMA] * (3 * NBUF),
    )
    def embed(x_hbm, tok_hbm, pos_hbm, out_hbm,
              idx_v, pos_sh, rows_v, *sems):
        wid = lax.axis_index("s") * 2 + lax.axis_index("c")
        seq0 = wid * spw
        isems = sems[:NBUF]
        osems = sems[NBUF:2 * NBUF]
        gsems = sems[2 * NBUF:3 * NBUF]

        @pl.when(lax.axis_index("s") == 0)
        def _():
            pltpu.sync_copy(pos_hbm, pos_sh)

        pltpu.sync_copy(x_hbm.at[pl.ds(seq0, spw)], idx_v)
        plsc.subcore_barrier()

        pltpu.async_copy(pos_sh, rows_v.at[0, :, pl.ds(0, D)], isems[0])

        def thin(slot):
            return rows_v.at[slot, :, pl.ds(0, D)]

        def out_thin(j):
            return out_hbm.at[seq0 + j, :, pl.ds(0, D)]

        def gwait(j, slot):
            buf = rows_v.at[slot]
            pltpu.make_async_copy(
                tok_hbm.at[idx_v.at[j, pl.ds(0, HALF_A)]],
                buf.at[pl.ds(0, HALF_A)], gsems[slot]).wait()
            pltpu.make_async_copy(
                tok_hbm.at[idx_v.at[j, pl.ds(HALF_A, HALF_B)]],
                buf.at[pl.ds(HALF_A, HALF_B)], gsems[slot]).wait()
            pltpu.async_copy(thin(slot), out_thin(j), osems[slot])

        def step(j, slot):
            buf = rows_v.at[slot]
            nslot = (slot + 1) % NBUF
            pltpu.make_async_copy(pos_sh, thin(slot), isems[slot]).wait()
            pltpu.async_copy(
                tok_hbm.at[idx_v.at[j, pl.ds(0, HALF_A)]],
                buf.at[pl.ds(0, HALF_A)], gsems[slot], add=True)
            pltpu.async_copy(
                tok_hbm.at[idx_v.at[j, pl.ds(HALF_A, HALF_B)]],
                buf.at[pl.ds(HALF_A, HALF_B)], gsems[slot], add=True)

            @pl.when(j >= 1)
            def _():
                gwait(j - 1, (slot - 1) % NBUF)

            @pl.when(j + 1 < spw)
            def _():
                @pl.when(j + 1 >= NBUF)
                def _():
                    pltpu.make_async_copy(thin(nslot), out_thin(j),
                                          osems[nslot]).wait()
                pltpu.async_copy(pos_sh, thin(nslot), isems[nslot])

        def outer(g, carry):
            for b in range(NBUF):
                step(g * NBUF + b, b)
            return carry

        lax.fori_loop(0, spw // NBUF, outer, 0)
        gwait(spw - 1, (spw - 1) % NBUF)
        for j in range(spw - NBUF, spw):
            pltpu.make_async_copy(
                thin(j % NBUF), out_thin(j), osems[j % NBUF]
            ).wait()

    out128 = embed(x, tok128, pos_table)
    return out128[:, :, :D]
